# Initial kernel scaffold; baseline (speedup 1.0000x reference)
import jax
import jax.numpy as jnp
from jax.experimental import pallas as pl


def kernel(T0, T1, T2, emb, W1, W2, Wout):
    raise NotImplementedError("write your pallas kernel here")


# speedup vs baseline: 4.0687x; 4.0687x over previous
import functools

import jax
import jax.numpy as jnp
from jax import lax
from jax.experimental import pallas as pl
from jax.experimental.pallas import tpu as pltpu
from jax.experimental.pallas import tpu_sc as plsc

D = 128
B = 1024
S1 = 32
S2 = 16
M1 = B * S1
NCLS = 64
NC, NS = 2, 16
NW = NC * NS
L = 16

CH = 128
G0_PW = B // NW
G1_PW = M1 // NW
G1_CHUNKS = G1_PW // CH
S2_OUT_PC = CH // S2
S2_CHUNKS = G1_PW // S2_OUT_PC


def _sc_gather(emb, t0, t1f, t2f):
    mesh = plsc.VectorSubcoreMesh(core_axis_name="c", subcore_axis_name="s")

    @functools.partial(
        pl.kernel,
        mesh=mesh,
        out_type=[
            jax.ShapeDtypeStruct((B, D), jnp.float32),
            jax.ShapeDtypeStruct((M1, D), jnp.float32),
            jax.ShapeDtypeStruct((M1, D), jnp.float32),
        ],
        scratch_types=[
            pltpu.VMEM((G0_PW,), jnp.int32),
            pltpu.VMEM((CH,), jnp.int32),
            pltpu.VMEM((CH, D), jnp.float32),
            pltpu.VMEM((S2_OUT_PC, D), jnp.float32),
            pltpu.SemaphoreType.DMA,
        ],
    )
    def k(emb_h, t0_h, t1_h, t2_h, g0_h, g1_h, s2_h,
          idx0_v, idx_v, rows_v, acc_v, sem):
        wid = lax.axis_index("s") * NC + lax.axis_index("c")

        b0 = wid * G0_PW
        pltpu.sync_copy(t0_h.at[pl.ds(b0, G0_PW)], idx0_v)
        pltpu.async_copy(emb_h.at[idx0_v], rows_v.at[pl.ds(0, G0_PW)], sem).wait()
        pltpu.sync_copy(rows_v.at[pl.ds(0, G0_PW)], g0_h.at[pl.ds(b0, G0_PW)])

        b1 = wid * G1_PW

        def g1_body(c, carry):
            off = b1 + c * CH
            pltpu.sync_copy(t1_h.at[pl.ds(off, CH)], idx_v)
            pltpu.async_copy(emb_h.at[idx_v], rows_v, sem).wait()
            pltpu.sync_copy(rows_v, g1_h.at[pl.ds(off, CH)])
            return carry

        lax.fori_loop(0, G1_CHUNKS, g1_body, 0)

        def s2_body(c, carry):
            orow = b1 + c * S2_OUT_PC
            pltpu.sync_copy(t2_h.at[pl.ds(orow * S2, CH)], idx_v)
            pltpu.async_copy(emb_h.at[idx_v], rows_v, sem).wait()

            def red_j(j, carry2):
                base = j * S2
                for lg in range(D // L):
                    sl = pl.ds(lg * L, L)
                    acc = rows_v[base, sl]
                    for s in range(1, S2):
                        acc = acc + rows_v[base + s, sl]
                    acc_v[j, sl] = acc
                return carry2

            lax.fori_loop(0, S2_OUT_PC, red_j, 0)
            pltpu.sync_copy(acc_v, s2_h.at[pl.ds(orow, S2_OUT_PC)])
            return carry

        lax.fori_loop(0, S2_CHUNKS, s2_body, 0)

    return k(emb, t0, t1f, t2f)


TCB = 2048
NSTEP = M1 // TCB
GPB = TCB // S1


def _tc_body(g0_r, g1_r, s2_r, w1a_r, w1b_r, w1bs_r, w2a_r, w2b_r, wout_r,
             out_r, x1_r, b1_r):
    i = pl.program_id(0)
    g1b = g1_r[...]
    h = jnp.dot(g1b, w1a_r[...], preferred_element_type=jnp.float32)
    h = h + jnp.dot(s2_r[...], w1bs_r[...], preferred_element_type=jnp.float32)
    h = jnp.maximum(h, 0.0)
    b1_r[pl.ds(i * GPB, GPB), :] = h.reshape(GPB, S1, D).sum(axis=1) * (1.0 / S1)
    x1_r[pl.ds(i * GPB, GPB), :] = g1b.reshape(GPB, S1, D).sum(axis=1) * (1.0 / S1)

    @pl.when(i == NSTEP - 1)
    def _():
        a = jnp.dot(g0_r[...], w1a_r[...], preferred_element_type=jnp.float32)
        a = a + jnp.dot(x1_r[...], w1b_r[...], preferred_element_type=jnp.float32)
        a = jnp.maximum(a, 0.0)
        c = jnp.dot(a, w2a_r[...], preferred_element_type=jnp.float32)
        c = c + jnp.dot(b1_r[...], w2b_r[...], preferred_element_type=jnp.float32)
        c = jnp.maximum(c, 0.0)
        out_r[...] = jnp.dot(c, wout_r[...], preferred_element_type=jnp.float32)


def _tc_dense(g0, g1, s2, w1a, w1b, w1bs, w2a, w2b, woutT):
    full = lambda shape: pl.BlockSpec(shape, lambda i: (0, 0))
    return pl.pallas_call(
        _tc_body,
        grid=(NSTEP,),
        in_specs=[
            full((B, D)),
            pl.BlockSpec((TCB, D), lambda i: (i, 0)),
            pl.BlockSpec((TCB, D), lambda i: (i, 0)),
            full((D, D)), full((D, D)), full((D, D)),
            full((D, D)), full((D, D)),
            full((D, NCLS)),
        ],
        out_specs=full((B, NCLS)),
        out_shape=jax.ShapeDtypeStruct((B, NCLS), jnp.float32),
        scratch_shapes=[
            pltpu.VMEM((B, D), jnp.float32),
            pltpu.VMEM((B, D), jnp.float32),
        ],
    )(g0, g1, s2, w1a, w1b, w1bs, w2a, w2b, woutT)


def kernel(T0, T1, T2, emb, W1, W2, Wout):
    t0 = T0.astype(jnp.int32)
    t1f = T1.reshape(-1).astype(jnp.int32)
    t2f = T2.reshape(-1).astype(jnp.int32)
    w1a = W1[:, :D].T
    w1b = W1[:, D:].T
    w1bs = w1b * (1.0 / S2)
    w2a = W2[:, :D].T
    w2b = W2[:, D:].T
    woutT = Wout.T
    g0, g1, s2 = _sc_gather(emb, t0, t1f, t2f)
    return _tc_dense(g0, g1, s2, w1a, w1b, w1bs, w2a, w2b, woutT)
